# baseline (speedup 1.0000x reference)
import functools

import jax
import jax.numpy as jnp
from jax import lax
from jax.experimental import pallas as pl
from jax.experimental.pallas import tpu as pltpu
from jax.experimental.pallas import tpu_sc as plsc

V = 1_000_000
D = 64
B = 327_680

HB = 256
GRID = -(-V // (2 * HB))
NP = GRID * HB


def _pack_kernel(a_ref, b_ref, out_ref):
    out_ref[...] = jnp.concatenate(
        [jnp.transpose(a_ref[...], (1, 0)),
         jnp.transpose(b_ref[...], (1, 0))], axis=1)


def _pack(table_t):
    return pl.pallas_call(
        _pack_kernel,
        grid=(GRID,),
        in_specs=[
            pl.BlockSpec((D, HB), lambda g: (0, jnp.minimum(2 * g, 2 * GRID - 2))),
            pl.BlockSpec((D, HB), lambda g: (0, jnp.minimum(2 * g + 1, 2 * GRID - 2))),
        ],
        out_specs=pl.BlockSpec((HB, 128), lambda g: (g, 0)),
        out_shape=jax.ShapeDtypeStruct((NP, 128), jnp.float32),
    )(table_t, table_t)


NC, NS = 2, 16
NW = NC * NS
CHUNK = 128
GROUP = 5
ROWS = CHUNK * GROUP
PER_W = B // NW
N_CHUNKS = PER_W // CHUNK
N_GROUPS = PER_W // ROWS
NBUF = 2
N_ROUNDS = N_GROUPS // NBUF


def _make_gather():
    mesh = plsc.VectorSubcoreMesh(
        core_axis_name="c", subcore_axis_name="s",
        num_cores=NC, num_subcores=NS)

    @functools.partial(
        pl.kernel,
        out_type=jax.ShapeDtypeStruct((B, D), jnp.float32),
        mesh=mesh,
        scratch_types=[
            pltpu.VMEM((N_CHUNKS, CHUNK), jnp.int32),
            pltpu.VMEM((NBUF, ROWS, D), jnp.float32),
            pltpu.SemaphoreType.DMA,
            pltpu.SemaphoreType.DMA,
            pltpu.SemaphoreType.DMA,
            pltpu.SemaphoreType.DMA,
        ],
        compiler_params=pltpu.CompilerParams(use_tc_tiling_on_sc=False),
    )
    def gather_kernel(idx_hbm, table_hbm, out_hbm, idx_v, rows_v,
                      gsem0, gsem1, wsem0, wsem1):
        wid = lax.axis_index("s") * NC + lax.axis_index("c")
        pltpu.sync_copy(idx_hbm.at[wid], idx_v)
        base = wid * PER_W
        gsem = (gsem0, gsem1)
        wsem = (wsem0, wsem1)

        def fire(g, b):
            for k in range(GROUP):
                pltpu.async_copy(
                    table_hbm.at[idx_v.at[g * GROUP + k]],
                    rows_v.at[b].at[pl.ds(k * CHUNK, CHUNK)],
                    gsem[b])

        def drain_gathers(b):
            for k in range(GROUP):
                pltpu.make_async_copy(
                    table_hbm.at[idx_v.at[0]],
                    rows_v.at[b].at[pl.ds(k * CHUNK, CHUNK)],
                    gsem[b]).wait()

        def start_write(g, b):
            pltpu.async_copy(
                rows_v.at[b], out_hbm.at[pl.ds(base + g * ROWS, ROWS)],
                wsem[b])

        def drain_write(b):
            pltpu.make_async_copy(
                rows_v.at[b], out_hbm.at[pl.ds(base, ROWS)], wsem[b]).wait()

        for b in range(NBUF):
            fire(b, b)

        @pl.loop(0, N_ROUNDS - 1)
        def body(r):
            g0 = r * NBUF
            for b in range(NBUF):
                drain_gathers(b)
                start_write(g0 + b, b)
            for b in range(NBUF):
                drain_write(b)
                fire(g0 + NBUF + b, b)

        for b in range(NBUF):
            drain_gathers(b)
            start_write((N_ROUNDS - 1) * NBUF + b, b)
        for b in range(NBUF):
            drain_write(b)

    return gather_kernel


_gather = _make_gather()


@jax.jit
def kernel(indices, table):
    r = indices.astype(jnp.int32)
    q = r & 511
    r2 = (r & ~jnp.int32(511)) + 2 * (q & 255) + (q >> 8)
    idx = r2.reshape(NW, N_CHUNKS, CHUNK)
    packed = _pack(table.T)
    flat = packed.reshape(2 * NP, D)
    out = _gather(idx, flat)
    return {"item_id": out}


# speedup vs baseline: 2.4565x; 2.4565x over previous
import functools

import jax
import jax.numpy as jnp
from jax import lax
from jax.experimental import pallas as pl
from jax.experimental.pallas import tpu as pltpu
from jax.experimental.pallas import tpu_sc as plsc

V = 1_000_000
D = 64
B = 327_680

HB = 2048
GRID = -(-V // (2 * HB))
NP = GRID * HB
MAXBLK = -(-V // HB) - 1


def _pack_kernel(a_ref, b_ref, out_ref):
    out_ref[...] = jnp.concatenate(
        [jnp.transpose(a_ref[...], (1, 0)),
         jnp.transpose(b_ref[...], (1, 0))], axis=1)


def _pack(table_t):
    return pl.pallas_call(
        _pack_kernel,
        grid=(GRID,),
        in_specs=[
            pl.BlockSpec((D, HB), lambda g: (0, jnp.minimum(2 * g, MAXBLK))),
            pl.BlockSpec((D, HB), lambda g: (0, jnp.minimum(2 * g + 1, MAXBLK))),
        ],
        out_specs=pl.BlockSpec((HB, 128), lambda g: (g, 0)),
        out_shape=jax.ShapeDtypeStruct((NP, 128), jnp.float32),
    )(table_t, table_t)


NC, NS = 2, 16
NW = NC * NS
CHUNK = 128
GROUP = 5
ROWS = CHUNK * GROUP
PER_W = B // NW
N_CHUNKS = PER_W // CHUNK
N_GROUPS = PER_W // ROWS
NBUF = 2
N_ROUNDS = N_GROUPS // NBUF


def _make_gather():
    mesh = plsc.VectorSubcoreMesh(
        core_axis_name="c", subcore_axis_name="s",
        num_cores=NC, num_subcores=NS)

    @functools.partial(
        pl.kernel,
        out_type=jax.ShapeDtypeStruct((B, D), jnp.float32),
        mesh=mesh,
        scratch_types=[
            pltpu.VMEM((N_CHUNKS, CHUNK), jnp.int32),
            pltpu.VMEM((NBUF, ROWS, D), jnp.float32),
            pltpu.SemaphoreType.DMA,
            pltpu.SemaphoreType.DMA,
            pltpu.SemaphoreType.DMA,
            pltpu.SemaphoreType.DMA,
        ],
        compiler_params=pltpu.CompilerParams(use_tc_tiling_on_sc=False),
    )
    def gather_kernel(idx_hbm, table_hbm, out_hbm, idx_v, rows_v,
                      gsem0, gsem1, wsem0, wsem1):
        wid = lax.axis_index("s") * NC + lax.axis_index("c")
        pltpu.sync_copy(idx_hbm.at[wid], idx_v)
        base = wid * PER_W
        gsem = (gsem0, gsem1)
        wsem = (wsem0, wsem1)

        def fire(g, b):
            for k in range(GROUP):
                pltpu.async_copy(
                    table_hbm.at[idx_v.at[g * GROUP + k]],
                    rows_v.at[b].at[pl.ds(k * CHUNK, CHUNK)],
                    gsem[b])

        def drain_gathers(b):
            for k in range(GROUP):
                pltpu.make_async_copy(
                    table_hbm.at[idx_v.at[0]],
                    rows_v.at[b].at[pl.ds(k * CHUNK, CHUNK)],
                    gsem[b]).wait()

        def start_write(g, b):
            pltpu.async_copy(
                rows_v.at[b], out_hbm.at[pl.ds(base + g * ROWS, ROWS)],
                wsem[b])

        def drain_write(b):
            pltpu.make_async_copy(
                rows_v.at[b], out_hbm.at[pl.ds(base, ROWS)], wsem[b]).wait()

        for b in range(NBUF):
            fire(b, b)

        @pl.loop(0, N_ROUNDS - 1)
        def body(r):
            g0 = r * NBUF
            for b in range(NBUF):
                drain_gathers(b)
                start_write(g0 + b, b)
            for b in range(NBUF):
                drain_write(b)
                fire(g0 + NBUF + b, b)

        for b in range(NBUF):
            drain_gathers(b)
            start_write((N_ROUNDS - 1) * NBUF + b, b)
        for b in range(NBUF):
            drain_write(b)

    return gather_kernel


_gather = _make_gather()


@jax.jit
def kernel(indices, table):
    r = indices.astype(jnp.int32)
    q = r & (2 * HB - 1)
    r2 = (r & ~jnp.int32(2 * HB - 1)) + 2 * (q & (HB - 1)) + (q // HB)
    idx = r2.reshape(NW, N_CHUNKS, CHUNK)
    packed = _pack(table.T)
    flat = packed.reshape(2 * NP, D)
    out = _gather(idx, flat)
    return {"item_id": out}
